# baseline (speedup 1.0000x reference)
import functools

import jax
import jax.numpy as jnp
from jax import lax
from jax.experimental import pallas as pl
from jax.experimental.pallas import tpu as pltpu
from jax.experimental.pallas import tpu_sc as plsc

_L = 16
_IR = 128


def _build(N, V, D, NW):
    C = 512
    CR = C // _IR
    rows_w = N // NW
    n_chunks = rows_w // C
    irows_w = rows_w // _IR

    mesh = plsc.VectorSubcoreMesh(core_axis_name="c", subcore_axis_name="s")

    @functools.partial(
        pl.kernel,
        mesh=mesh,
        compiler_params=pltpu.CompilerParams(use_tc_tiling_on_sc=False),
        out_type=jax.ShapeDtypeStruct((N, D), jnp.float32),
        scratch_types=[
            pltpu.VMEM((CR, _IR), jnp.int32),
            pltpu.VMEM((CR, _IR), jnp.int32),
            pltpu.VMEM((C, D), jnp.float32),
            pltpu.SemaphoreType.DMA,
        ],
    )
    def k(xf_hbm, table_hbm, out_hbm, xbuf, idxbuf, rowbuf, sem):
        nc = lax.axis_size("c")
        wid = lax.axis_index("s") * nc + lax.axis_index("c")
        irow0 = wid * irows_w

        def body(ci, carry):
            r0 = irow0 + ci * CR
            pltpu.sync_copy(xf_hbm.at[pl.ds(r0, CR)], xbuf)
            for r in range(CR):
                for v in range(_IR // _L):
                    s = pl.ds(v * _L, _L)
                    idxbuf[r, s] = -xbuf[r, s] - 1
            for j in range(CR):
                pltpu.async_copy(
                    table_hbm.at[idxbuf.at[j]],
                    rowbuf.at[pl.ds(j * _IR, _IR)],
                    sem,
                )
            for j in range(CR):
                pltpu.make_async_copy(
                    table_hbm.at[idxbuf.at[j]],
                    rowbuf.at[pl.ds(j * _IR, _IR)],
                    sem,
                ).wait()
            pltpu.sync_copy(rowbuf, out_hbm.at[pl.ds(r0 * _IR, C)])
            return carry

        lax.fori_loop(0, n_chunks, body, 0)

    return k


def kernel(x, table):
    B, H = x.shape
    V, D = table.shape
    N = B * H
    info = plsc.get_sparse_core_info()
    NW = info.num_cores * info.num_subcores
    xf = x.reshape(N // _IR, _IR).astype(jnp.int32)
    out = _build(N, V, D, NW)(xf, table)
    return out.reshape(B, H, D)


# speedup vs baseline: 1.0787x; 1.0787x over previous
import functools

import jax
import jax.numpy as jnp
from jax import lax
from jax.experimental import pallas as pl
from jax.experimental.pallas import tpu as pltpu
from jax.experimental.pallas import tpu_sc as plsc

_L = 16
_IR = 128


def _build(N, V, D, NW):
    C = 512
    CR = C // _IR
    rows_w = N // NW
    n_chunks = rows_w // C
    n_pairs = n_chunks // 2
    irows_w = rows_w // _IR

    mesh = plsc.VectorSubcoreMesh(core_axis_name="c", subcore_axis_name="s")

    @functools.partial(
        pl.kernel,
        mesh=mesh,
        compiler_params=pltpu.CompilerParams(use_tc_tiling_on_sc=False),
        out_type=jax.ShapeDtypeStruct((N, D), jnp.float32),
        scratch_types=[
            pltpu.VMEM((2, CR, _IR), jnp.int32),
            pltpu.VMEM((2, CR, _IR), jnp.int32),
            pltpu.VMEM((2, C, D), jnp.float32),
            pltpu.SemaphoreType.DMA,
            pltpu.SemaphoreType.DMA,
            pltpu.SemaphoreType.DMA,
            pltpu.SemaphoreType.DMA,
        ],
    )
    def k(xf_hbm, table_hbm, out_hbm, xbuf, idxbuf, rowbuf, g0, g1, o0, o1):
        nc = lax.axis_size("c")
        wid = lax.axis_index("s") * nc + lax.axis_index("c")
        irow0 = wid * irows_w
        gsem = (g0, g1)
        osem = (o0, o1)

        def fire_gathers(slot, g):
            r0 = irow0 + g * CR
            pltpu.sync_copy(xf_hbm.at[pl.ds(r0, CR)], xbuf.at[slot])
            for r in range(CR):
                for v in range(_IR // _L):
                    s = pl.ds(v * _L, _L)
                    idxbuf[slot, r, s] = -xbuf[slot, r, s] - 1
            for j in range(CR):
                pltpu.async_copy(
                    table_hbm.at[idxbuf.at[slot].at[j]],
                    rowbuf.at[slot].at[pl.ds(j * _IR, _IR)],
                    gsem[slot],
                )

        def drain_gathers(slot):
            for j in range(CR):
                pltpu.make_async_copy(
                    table_hbm.at[idxbuf.at[slot].at[j]],
                    rowbuf.at[slot].at[pl.ds(j * _IR, _IR)],
                    gsem[slot],
                ).wait()

        def out_copy(slot, g):
            r0 = irow0 + g * CR
            return pltpu.make_async_copy(
                rowbuf.at[slot], out_hbm.at[pl.ds(r0 * _IR, C)], osem[slot]
            )

        def body(p, carry):
            gc0 = 2 * p

            @pl.when(p >= 1)
            def _finish_prev_slot1():
                drain_gathers(1)
                out_copy(1, gc0 - 1).start()
                out_copy(0, gc0 - 2).wait()

            fire_gathers(0, gc0)

            @pl.when(p >= 1)
            def _free_slot1():
                out_copy(1, gc0 - 1).wait()

            fire_gathers(1, gc0 + 1)
            drain_gathers(0)
            out_copy(0, gc0).start()
            return carry

        lax.fori_loop(0, n_pairs, body, 0)
        drain_gathers(1)
        out_copy(1, n_chunks - 1).start()
        out_copy(0, n_chunks - 2).wait()
        out_copy(1, n_chunks - 1).wait()

    return k


def kernel(x, table):
    B, H = x.shape
    V, D = table.shape
    N = B * H
    info = plsc.get_sparse_core_info()
    NW = info.num_cores * info.num_subcores
    xf = x.reshape(N // _IR, _IR).astype(jnp.int32)
    out = _build(N, V, D, NW)(xf, table)
    return out.reshape(B, H, D)
